# baseline (speedup 1.0000x reference)
import jax
import jax.numpy as jnp
from jax import lax
from jax.experimental import pallas as pl
from jax.experimental.pallas import tpu as pltpu
from jax.experimental.pallas import tpu_sc as plsc

B, L, C, D = 4096, 20, 16, 64
V, CV = 1000000, 1000
HALFV = V // 2

NC, NS = 2, 16
NW = NC * NS
RPW = B // NW
BLK = 4
NBLK = RPW // BLK
IDX_PER_BLK = BLK * L
LC = L * C

TCOLS = 1024
TGRID = -(-V // TCOLS)
VLIN = TGRID * TCOLS


def _pack_char_table(char_table):
    t = char_table.astype(jnp.bfloat16).reshape(CV, 2, 2, 16)
    t = t.transpose(0, 1, 3, 2).reshape(CV, 32, 2)
    return lax.bitcast_convert_type(t, jnp.int32)


def _tc_transpose_body(in_ref, out_ref):
    t = in_ref[...].T
    out_ref[...] = jnp.concatenate(
        [t[0:TCOLS // 2, :], t[TCOLS // 2:TCOLS, :]], axis=1)


def _linearize_table(word_table):
    wt_t = word_table.T
    out = pl.pallas_call(
        _tc_transpose_body,
        grid=(TGRID,),
        in_specs=[pl.BlockSpec((D, TCOLS), lambda c: (0, c))],
        out_specs=pl.BlockSpec((TCOLS // 2, 2 * D), lambda c: (c, 0)),
        out_shape=jax.ShapeDtypeStruct((VLIN // 2, 2 * D), jnp.float32),
    )(wt_t)
    return out.reshape(VLIN, D)


def _char_body(xc_hbm, ctab_hbm, out_hbm, ctab_v, xcidx_v, out_v):
    wid = lax.axis_index("s") * NC + lax.axis_index("c")
    row0 = wid * RPW

    pltpu.sync_copy(ctab_hbm, ctab_v)
    pltpu.sync_copy(xc_hbm.at[pl.ds(row0, RPW)], xcidx_v)

    c320 = jnp.full((16,), jnp.float32(1.0 / 320.0))
    sh16 = jnp.full((16,), 16, dtype=jnp.int32)
    zero = jnp.zeros((16,), jnp.float32)

    def char_row(r, carry):
        def cbody(it, accs):
            a0, a1, a2, a3 = accs
            iv = xcidx_v[r, pl.ds(it * 16, 16)]
            for k in range(16):
                v = iv[k]
                w0 = ctab_v[v, pl.ds(0, 16)]
                w1 = ctab_v[v, pl.ds(16, 16)]
                a0 = a0 + lax.bitcast_convert_type(
                    lax.shift_left(w0, sh16), jnp.float32)
                a1 = a1 + lax.bitcast_convert_type(w0, jnp.float32)
                a2 = a2 + lax.bitcast_convert_type(
                    lax.shift_left(w1, sh16), jnp.float32)
                a3 = a3 + lax.bitcast_convert_type(w1, jnp.float32)
            return a0, a1, a2, a3

        a0, a1, a2, a3 = lax.fori_loop(
            0, LC // 16, cbody, (zero, zero, zero, zero))
        out_v[r, pl.ds(0, 16)] = a0 * c320
        out_v[r, pl.ds(16, 16)] = a1 * c320
        out_v[r, pl.ds(32, 16)] = a2 * c320
        out_v[r, pl.ds(48, 16)] = a3 * c320
        return carry

    lax.fori_loop(0, RPW, char_row, 0)
    pltpu.sync_copy(out_v, out_hbm.at[pl.ds(row0, RPW)])


def _word_body(xr_hbm, wt_hbm, out_hbm, widx_v, g0_v, g1_v, out_v, sem0, sem1):
    wid = lax.axis_index("s") * NC + lax.axis_index("c")
    row0 = wid * RPW

    pltpu.sync_copy(xr_hbm.at[pl.ds(wid * NBLK, NBLK)], widx_v)

    cm1024 = jnp.full((16,), -1024, dtype=jnp.int32)
    c511 = jnp.full((16,), 511, dtype=jnp.int32)
    c9 = jnp.full((16,), 9, dtype=jnp.int32)
    c1i = jnp.full((16,), 1, dtype=jnp.int32)

    def remap(t, carry):
        r = t // (IDX_PER_BLK // 16)
        o = (t % (IDX_PER_BLK // 16)) * 16
        v = widx_v[r, pl.ds(o, 16)]
        lo = v & c511
        widx_v[r, pl.ds(o, 16)] = (
            (v & cm1024) + lo + lo + (lax.shift_right_logical(v, c9) & c1i))
        return carry

    lax.fori_loop(0, NBLK * (IDX_PER_BLK // 16), remap, 0)

    c20 = jnp.full((16,), jnp.float32(1.0 / 20.0))
    zero = jnp.zeros((16,), jnp.float32)

    def fire(b, gbuf, sem):
        return pltpu.async_copy(wt_hbm.at[widx_v.at[b]], gbuf, sem)

    def accum(b, gbuf):
        def word_row(i, carry):
            def wbody(l, accs):
                a0, a1, a2, a3 = accs
                g = i * L + l
                a0 = a0 + gbuf[g, pl.ds(0, 16)]
                a1 = a1 + gbuf[g, pl.ds(16, 16)]
                a2 = a2 + gbuf[g, pl.ds(32, 16)]
                a3 = a3 + gbuf[g, pl.ds(48, 16)]
                return a0, a1, a2, a3

            a0, a1, a2, a3 = lax.fori_loop(
                0, L, wbody, (zero, zero, zero, zero), unroll=4)
            r = b * BLK + i
            out_v[r, pl.ds(0, 16)] = a0 * c20
            out_v[r, pl.ds(16, 16)] = a1 * c20
            out_v[r, pl.ds(32, 16)] = a2 * c20
            out_v[r, pl.ds(48, 16)] = a3 * c20
            return carry

        lax.fori_loop(0, BLK, word_row, 0)

    fire(0, g0_v, sem0)

    def block2(t, carry):
        b0 = t * 2
        fire(b0 + 1, g1_v, sem1)
        pltpu.make_async_copy(wt_hbm.at[widx_v.at[b0]], g0_v, sem0).wait()
        accum(b0, g0_v)

        @pl.when(t < NBLK // 2 - 1)
        def _():
            fire(b0 + 2, g0_v, sem0)

        pltpu.make_async_copy(wt_hbm.at[widx_v.at[b0 + 1]], g1_v, sem1).wait()
        accum(b0 + 1, g1_v)
        return carry

    lax.fori_loop(0, NBLK // 2, block2, 0)
    pltpu.sync_copy(out_v, out_hbm.at[pl.ds(row0, RPW)])


@jax.jit
def kernel(x, x_char, word_table, char_table):
    xr = x.reshape(B * L // IDX_PER_BLK, IDX_PER_BLK)
    xc = x_char.reshape(B, LC)
    ctab = _pack_char_table(char_table)
    wt_lin = _linearize_table(word_table)

    mesh = plsc.VectorSubcoreMesh(core_axis_name="c", subcore_axis_name="s")
    sc_params = pltpu.CompilerParams(use_tc_tiling_on_sc=False)

    char_run = pl.kernel(
        _char_body,
        mesh=mesh,
        compiler_params=sc_params,
        out_type=jax.ShapeDtypeStruct((B, D), jnp.float32),
        scratch_types=[
            pltpu.VMEM((CV, 32), jnp.int32),
            pltpu.VMEM((RPW, LC), jnp.int32),
            pltpu.VMEM((RPW, D), jnp.float32),
        ],
    )
    word_run = pl.kernel(
        _word_body,
        mesh=mesh,
        compiler_params=sc_params,
        out_type=jax.ShapeDtypeStruct((B, D), jnp.float32),
        scratch_types=[
            pltpu.VMEM((NBLK, IDX_PER_BLK), jnp.int32),
            pltpu.VMEM((IDX_PER_BLK, D), jnp.float32),
            pltpu.VMEM((IDX_PER_BLK, D), jnp.float32),
            pltpu.VMEM((RPW, D), jnp.float32),
            pltpu.SemaphoreType.DMA,
            pltpu.SemaphoreType.DMA,
        ],
    )
    cboc = char_run(xc, ctab)
    cbow = word_run(xr, wt_lin)
    return jnp.concatenate([cbow, cboc], axis=1)
